# baseline (speedup 1.0000x reference)
import functools

import jax
import jax.numpy as jnp
from jax import lax
from jax.experimental import pallas as pl
from jax.experimental.pallas import tpu as pltpu
from jax.experimental.pallas import tpu_sc as plsc

_ND = 8
_NB = 4
_SW = 512
_D = 4096


def _positional_lookup(table8, idx):
    n_rows = table8.shape[1] // _SW
    b = idx.shape[0]
    bpg = b // _NB
    n_vecs = bpg // 16
    mesh = plsc.VectorSubcoreMesh(core_axis_name="c", subcore_axis_name="s")

    @functools.partial(
        pl.kernel,
        mesh=mesh,
        out_type=jax.ShapeDtypeStruct((b * _D,), jnp.float32),
        scratch_types=[
            pltpu.VMEM((bpg,), jnp.int32),
            pltpu.VMEM((table8.shape[1],), jnp.float32),
            pltpu.SemaphoreType.DMA,
        ],
    )
    def k(idx_hbm, tab_hbm, out_hbm, idx_v, tab_tile, wsem):
        sid = lax.axis_index("s")
        cid = lax.axis_index("c")
        dgrp = lax.rem(sid, _ND)
        bgrp = lax.div(sid, _ND) * 2 + cid
        bbase = bgrp * bpg
        pltpu.sync_copy(idx_hbm.at[pl.ds(bbase, bpg)], idx_v)
        pltpu.sync_copy(tab_hbm.at[dgrp], tab_tile)
        colw = dgrp * _SW

        def vec_body(g, carry):
            vec = idx_v[pl.ds(g * 16, 16)]
            for l in range(16):
                s = vec[l]
                row = bbase + g * 16 + l
                pltpu.make_async_copy(
                    tab_tile.at[pl.ds(s * _SW, _SW)],
                    out_hbm.at[pl.ds(row * _D + colw, _SW)],
                    wsem,
                ).start()
            return carry

        lax.fori_loop(0, n_vecs, vec_body, 0)

        def drain_body(g, carry):
            pltpu.make_async_copy(
                out_hbm.at[pl.ds(0, 16 * _SW)],
                tab_tile.at[pl.ds(0, 16 * _SW)],
                wsem,
            ).wait()
            return carry

        lax.fori_loop(0, n_vecs, drain_body, 0)

    return k(idx, table8)


def kernel(x, weights):
    n_rows = weights.shape[0]
    d = weights.shape[1] * weights.shape[2]
    table = weights.reshape(n_rows, d)
    pad = (-n_rows) % 8
    if pad:
        table = jnp.pad(table, ((0, pad), (0, 0)))
    nr = table.shape[0]
    table8 = table.reshape(nr, _ND, _SW).transpose(1, 0, 2).reshape(_ND, nr * _SW)
    out = _positional_lookup(table8, x)
    return out.reshape(x.shape[0], d)
